# Initial kernel scaffold; baseline (speedup 1.0000x reference)
import jax
import jax.numpy as jnp
from jax.experimental import pallas as pl


def kernel(path_features, link_features, eps, path_indices, link_indices, W_path_in, b_path_in, W_link_in, b_link_in, Wi_p, Wh_p, bi_p, bh_p, Wi_l, Wh_l, bi_l, bh_l, W_mu, b_mu, W_lv, b_lv, W_r1, b_r1, W_r2, b_r2, W_r3, b_r3):
    raise NotImplementedError("write your pallas kernel here")


# speedup vs baseline: 1.0677x; 1.0677x over previous
import functools

import jax
import jax.numpy as jnp
from jax.experimental import pallas as pl

N_PATHS = 10000
N_LINKS = 5000
N_PL = 320000
HID = 128
LAT = 32
T = 8

_ROW_BLK = 1000


def _embed_body(x_ref, w_ref, b_ref, o_ref):
    o_ref[...] = jnp.tanh(
        jnp.dot(x_ref[...], w_ref[...], preferred_element_type=jnp.float32)
        + b_ref[...])


def _embed(x, w, b):
    n, d = x.shape
    p = w.shape[1]
    return pl.pallas_call(
        _embed_body,
        out_shape=jax.ShapeDtypeStruct((n, p), jnp.float32),
        grid=(n // _ROW_BLK,),
        in_specs=[
            pl.BlockSpec((_ROW_BLK, d), lambda i: (i, 0)),
            pl.BlockSpec((d, p), lambda i: (0, 0)),
            pl.BlockSpec((1, p), lambda i: (0, 0)),
        ],
        out_specs=pl.BlockSpec((_ROW_BLK, p), lambda i: (i, 0)),
    )(x, w, b.reshape(1, -1))


def _gru_body(m_ref, h_ref, wi_ref, wh_ref, bi_ref, bh_ref, o_ref):
    m = m_ref[...]
    h = h_ref[...]
    gi = jnp.dot(m, wi_ref[...], preferred_element_type=jnp.float32) + bi_ref[...]
    gh = jnp.dot(h, wh_ref[...], preferred_element_type=jnp.float32) + bh_ref[...]
    H = h.shape[1]
    r = jax.nn.sigmoid(gi[:, :H] + gh[:, :H])
    z = jax.nn.sigmoid(gi[:, H:2 * H] + gh[:, H:2 * H])
    n = jnp.tanh(gi[:, 2 * H:] + r * gh[:, 2 * H:])
    o_ref[...] = (1.0 - z) * n + z * h


def _gru(m, h, wi, wh, bi, bh):
    n, H = h.shape
    return pl.pallas_call(
        _gru_body,
        out_shape=jax.ShapeDtypeStruct((n, H), jnp.float32),
        grid=(n // _ROW_BLK,),
        in_specs=[
            pl.BlockSpec((_ROW_BLK, H), lambda i: (i, 0)),
            pl.BlockSpec((_ROW_BLK, H), lambda i: (i, 0)),
            pl.BlockSpec((H, 3 * H), lambda i: (0, 0)),
            pl.BlockSpec((H, 3 * H), lambda i: (0, 0)),
            pl.BlockSpec((1, 3 * H), lambda i: (0, 0)),
            pl.BlockSpec((1, 3 * H), lambda i: (0, 0)),
        ],
        out_specs=pl.BlockSpec((_ROW_BLK, H), lambda i: (i, 0)),
    )(m, h, wi, wh, bi.reshape(1, -1), bh.reshape(1, -1))


def _readout_body(h_ref, eps_ref, wmu_ref, bmu_ref, wlv_ref, blv_ref,
                  w1_ref, b1_ref, w2_ref, b2_ref, w3_ref, b3_ref, o_ref):
    h = h_ref[...]
    mu = jnp.dot(h, wmu_ref[...], preferred_element_type=jnp.float32) + bmu_ref[...]
    lv = jnp.dot(h, wlv_ref[...], preferred_element_type=jnp.float32) + blv_ref[...]
    z = eps_ref[...] * jnp.exp(0.5 * lv) + mu
    a = jnp.maximum(
        jnp.dot(z, w1_ref[...], preferred_element_type=jnp.float32) + b1_ref[...], 0.0)
    a = jnp.maximum(
        jnp.dot(a, w2_ref[...], preferred_element_type=jnp.float32) + b2_ref[...], 0.0)
    o_ref[...] = jnp.dot(a, w3_ref[...], preferred_element_type=jnp.float32) + b3_ref[...]


def _readout(h, eps, wmu, bmu, wlv, blv, w1, b1, w2, b2, w3, b3):
    n = h.shape[0]
    w3p = jnp.zeros((w3.shape[0], 128), jnp.float32).at[:, :1].set(w3)
    b3p = jnp.zeros((128,), jnp.float32).at[0].set(b3[0])
    out = pl.pallas_call(
        _readout_body,
        out_shape=jax.ShapeDtypeStruct((n, 128), jnp.float32),
        grid=(n // _ROW_BLK,),
        in_specs=[
            pl.BlockSpec((_ROW_BLK, h.shape[1]), lambda i: (i, 0)),
            pl.BlockSpec((_ROW_BLK, eps.shape[1]), lambda i: (i, 0)),
        ] + [pl.BlockSpec(w.shape, lambda i: (0, 0)) for w in
             (wmu, bmu.reshape(1, -1), wlv, blv.reshape(1, -1),
              w1, b1.reshape(1, -1), w2, b2.reshape(1, -1),
              w3p, b3p.reshape(1, -1))],
        out_specs=pl.BlockSpec((_ROW_BLK, 128), lambda i: (i, 0)),
    )(h, eps, wmu, bmu.reshape(1, -1), wlv, blv.reshape(1, -1),
      w1, b1.reshape(1, -1), w2, b2.reshape(1, -1), w3p, b3p.reshape(1, -1))
    return out[:, 0]


def kernel(path_features, link_features, eps, path_indices, link_indices,
           W_path_in, b_path_in, W_link_in, b_link_in,
           Wi_p, Wh_p, bi_p, bh_p, Wi_l, Wh_l, bi_l, bh_l,
           W_mu, b_mu, W_lv, b_lv, W_r1, b_r1, W_r2, b_r2, W_r3, b_r3):
    dp = path_features.shape[1]
    dl = link_features.shape[1]
    pf = jnp.zeros((N_PATHS, 128), jnp.float32).at[:, :dp].set(path_features)
    lf = jnp.zeros((N_LINKS, 128), jnp.float32).at[:, :dl].set(link_features)
    wp = jnp.zeros((128, 128), jnp.float32).at[:dp].set(W_path_in)
    wl = jnp.zeros((128, 128), jnp.float32).at[:dl].set(W_link_in)

    h_path = _embed(pf, wp, b_path_in)
    h_link = _embed(lf, wl, b_link_in)

    for _ in range(T):
        m_path = jax.ops.segment_sum(h_link[link_indices], path_indices,
                                     num_segments=N_PATHS)
        h_path = _gru(m_path, h_path, Wi_p, Wh_p, bi_p, bh_p)
        m_link = jax.ops.segment_sum(h_path[path_indices], link_indices,
                                     num_segments=N_LINKS)
        h_link = _gru(m_link, h_link, Wi_l, Wh_l, bi_l, bh_l)

    return _readout(h_path, eps, W_mu, b_mu, W_lv, b_lv,
                    W_r1, b_r1, W_r2, b_r2, W_r3, b_r3)


# speedup vs baseline: 2.0222x; 1.8940x over previous
import numpy as np
import jax
import jax.numpy as jnp
from jax import lax
from jax.experimental import pallas as pl
from jax.experimental.pallas import tpu as pltpu
from jax.experimental.pallas import tpu_sc as plsc

N_PATHS = 10000
N_LINKS = 5000
N_PL = 320000
HID = 128
T = 8

NP_PAD = 10240
NL_PAD = 5120
CL = 128
CHUNKS = 80
N_TILES = 32
EPT = CHUNKS * CL
W = 688

_HALF = [10080] * 11 + [9840] * 4 + [9760]
WINDOW_SIZES = np.array(_HALF + _HALF, np.int32)
WINDOW_STARTS = np.concatenate([[0], np.cumsum(WINDOW_SIZES)[:-1]]).astype(np.int32)
assert int(WINDOW_SIZES.sum()) == N_PL

_ROW_BLK = 1024


def _make_seg_sum(n_dst_pad, sent):
    mesh = plsc.VectorSubcoreMesh(core_axis_name="c", subcore_axis_name="s")

    def body(table_hbm, sidx_hbm, didx_hbm, out_hbm, bndr_hbm, bndi_hbm,
             sidx_v, didx_v, rows_v, dense_v, nb_v, sem):
        c = lax.axis_index("c")
        s = lax.axis_index("s")
        wid = c * 16 + s
        pltpu.sync_copy(sidx_hbm.at[wid], sidx_v)
        pltpu.sync_copy(didx_hbm.at[wid], didx_v.at[pl.ds(0, EPT)])

        zero16 = jnp.zeros((16,), jnp.float32)
        lane = lax.iota(jnp.int32, 16)

        @pl.loop(0, W * (HID // 16))
        def _zero(r):
            dense_v[pl.ds(r * 16, 16)] = zero16

        base0 = didx_v[pl.ds(0, 16)][0]
        base = jnp.where(wid == 0, 0, base0)

        @pl.when(wid < N_TILES - 1)
        def _load_next():
            pltpu.sync_copy(didx_hbm.at[wid + 1].at[pl.ds(0, 16)], nb_v)

        nxt = nb_v[pl.ds(0, 16)][0]
        range_end = jnp.where(wid == N_TILES - 1, n_dst_pad, nxt)
        span = range_end - base

        def fold_chunk(q, carry):
            pltpu.make_async_copy(table_hbm.at[sidx_v.at[q]], rows_v, sem).wait()

            def edge(e, carry):
                cur, acc = carry
                d = didx_v[pl.ds(q * CL + e, 16)][0]
                is_new = d != cur

                @pl.when(jnp.logical_and(is_new, cur >= 0))
                def _store_run():
                    slot = jnp.minimum(jnp.maximum(cur - base, 0), W - 1) * HID
                    for v in range(HID // 16):
                        dense_v[pl.ds(slot + v * 16, 16)] = acc[v]

                acc2 = tuple(
                    jnp.where(is_new, rows_v[e, pl.ds(v * 16, 16)],
                              acc[v] + rows_v[e, pl.ds(v * 16, 16)])
                    for v in range(HID // 16))
                return (d, acc2)

            carry = pl.loop(0, CL, init_carry=carry)(edge)

            @pl.when(q < CHUNKS - 1)
            def _fire_next():
                pltpu.async_copy(table_hbm.at[sidx_v.at[q + 1]], rows_v, sem)

            return carry

        carry0 = (jnp.int32(-1), tuple(zero16 for _ in range(8)))
        pltpu.async_copy(table_hbm.at[sidx_v.at[0]], rows_v, sem)
        pl.loop(0, CHUNKS, init_carry=carry0)(fold_chunk)

        nblk = span // 16

        @pl.loop(0, nblk)
        def _wb(k):
            pltpu.sync_copy(dense_v.at[pl.ds(k * 16 * HID, 16 * HID)],
                            out_hbm.at[pl.ds((base + k * 16) * HID, 16 * HID)])

        @pl.loop(nblk * 16, span)
        def _wb_tail(r):
            pltpu.sync_copy(dense_v.at[pl.ds(r * HID, HID)],
                            out_hbm.at[pl.ds((base + r) * HID, HID)])

        bslot = jnp.minimum(span, W - 1) * HID
        pltpu.sync_copy(dense_v.at[pl.ds(bslot, HID)], bndr_hbm.at[wid])
        nb_v[pl.ds(0, 16)] = jnp.where(lane == 0,
                                       jnp.full((16,), range_end, jnp.int32),
                                       jnp.zeros((16,), jnp.int32))
        pltpu.sync_copy(nb_v, bndi_hbm.at[wid].at[pl.ds(0, 16)])

    return pl.kernel(
        body,
        out_type=(jax.ShapeDtypeStruct((n_dst_pad * HID,), jnp.float32),
                  jax.ShapeDtypeStruct((N_TILES, HID), jnp.float32),
                  jax.ShapeDtypeStruct((N_TILES, 16), jnp.int32)),
        mesh=mesh,
        scratch_types=[
            pltpu.VMEM((CHUNKS, CL), jnp.int32),
            pltpu.VMEM((EPT + 16,), jnp.int32),
            pltpu.VMEM((CL, HID), jnp.float32),
            pltpu.VMEM((W * HID,), jnp.float32),
            pltpu.VMEM((16,), jnp.int32),
            pltpu.SemaphoreType.DMA,
        ],
    )


_seg_sum_paths = _make_seg_sum(NP_PAD, N_PATHS)
_seg_sum_links = _make_seg_sum(NL_PAD, N_LINKS)


def _tile_edges(src, dst, sent):
    col = np.arange(EPT, dtype=np.int32)[None, :]
    valid = jnp.asarray(col < WINDOW_SIZES[:, None])
    gidx = jnp.asarray(np.minimum(WINDOW_STARTS[:, None] + col, N_PL - 1))
    src_t = jnp.where(valid, src[gidx], 0).reshape(N_TILES, CHUNKS, CL)
    dst_t = jnp.where(valid, dst[gidx], sent).reshape(N_TILES, EPT)
    return src_t, dst_t


def _bnd_body(m_ref, bndr_ref, bndi_ref, o_ref):
    i = pl.program_id(0)

    @pl.when(i == 0)
    def _init():
        o_ref[...] = m_ref[...]

    rid = bndi_ref[0, i]

    @pl.when(rid < o_ref.shape[0])
    def _add():
        o_ref[pl.ds(rid, 1), :] = o_ref[pl.ds(rid, 1), :] + bndr_ref[pl.ds(i, 1), :]


def _bnd_combine(m, bndr, bndi):
    n = m.shape[0]
    return pl.pallas_call(
        _bnd_body,
        out_shape=jax.ShapeDtypeStruct((n, HID), jnp.float32),
        grid=(N_TILES,),
        in_specs=[
            pl.BlockSpec((n, HID), lambda i: (0, 0)),
            pl.BlockSpec((N_TILES, HID), lambda i: (0, 0)),
            pl.BlockSpec(memory_space=pltpu.SMEM),
        ],
        out_specs=pl.BlockSpec((n, HID), lambda i: (0, 0)),
    )(m, bndr, bndi)


def _embed_body(x_ref, w_ref, b_ref, o_ref):
    o_ref[...] = jnp.tanh(
        jnp.dot(x_ref[...], w_ref[...], preferred_element_type=jnp.float32)
        + b_ref[...])


def _embed(x, w, b):
    n, d = x.shape
    p = w.shape[1]
    return pl.pallas_call(
        _embed_body,
        out_shape=jax.ShapeDtypeStruct((n, p), jnp.float32),
        grid=(n // _ROW_BLK,),
        in_specs=[
            pl.BlockSpec((_ROW_BLK, d), lambda i: (i, 0)),
            pl.BlockSpec((d, p), lambda i: (0, 0)),
            pl.BlockSpec((1, p), lambda i: (0, 0)),
        ],
        out_specs=pl.BlockSpec((_ROW_BLK, p), lambda i: (i, 0)),
    )(x, w, b.reshape(1, -1))


def _gru_body(m_ref, h_ref, wi_ref, wh_ref, bi_ref, bh_ref, o_ref):
    m = m_ref[...]
    h = h_ref[...]
    gi = jnp.dot(m, wi_ref[...], preferred_element_type=jnp.float32) + bi_ref[...]
    gh = jnp.dot(h, wh_ref[...], preferred_element_type=jnp.float32) + bh_ref[...]
    H = h.shape[1]
    r = jax.nn.sigmoid(gi[:, :H] + gh[:, :H])
    z = jax.nn.sigmoid(gi[:, H:2 * H] + gh[:, H:2 * H])
    n = jnp.tanh(gi[:, 2 * H:] + r * gh[:, 2 * H:])
    o_ref[...] = (1.0 - z) * n + z * h


def _gru(m, h, wi, wh, bi, bh):
    n, H = h.shape
    return pl.pallas_call(
        _gru_body,
        out_shape=jax.ShapeDtypeStruct((n, H), jnp.float32),
        grid=(n // _ROW_BLK,),
        in_specs=[
            pl.BlockSpec((_ROW_BLK, H), lambda i: (i, 0)),
            pl.BlockSpec((_ROW_BLK, H), lambda i: (i, 0)),
            pl.BlockSpec((H, 3 * H), lambda i: (0, 0)),
            pl.BlockSpec((H, 3 * H), lambda i: (0, 0)),
            pl.BlockSpec((1, 3 * H), lambda i: (0, 0)),
            pl.BlockSpec((1, 3 * H), lambda i: (0, 0)),
        ],
        out_specs=pl.BlockSpec((_ROW_BLK, H), lambda i: (i, 0)),
    )(m, h, wi, wh, bi.reshape(1, -1), bh.reshape(1, -1))


def _readout_body(h_ref, eps_ref, wmu_ref, bmu_ref, wlv_ref, blv_ref,
                  w1_ref, b1_ref, w2_ref, b2_ref, w3_ref, b3_ref, o_ref):
    h = h_ref[...]
    mu = jnp.dot(h, wmu_ref[...], preferred_element_type=jnp.float32) + bmu_ref[...]
    lv = jnp.dot(h, wlv_ref[...], preferred_element_type=jnp.float32) + blv_ref[...]
    z = eps_ref[...] * jnp.exp(0.5 * lv) + mu
    a = jnp.maximum(
        jnp.dot(z, w1_ref[...], preferred_element_type=jnp.float32) + b1_ref[...], 0.0)
    a = jnp.maximum(
        jnp.dot(a, w2_ref[...], preferred_element_type=jnp.float32) + b2_ref[...], 0.0)
    o_ref[...] = jnp.dot(a, w3_ref[...], preferred_element_type=jnp.float32) + b3_ref[...]


def _readout(h, eps, wmu, bmu, wlv, blv, w1, b1, w2, b2, w3, b3):
    n = h.shape[0]
    w3p = jnp.zeros((w3.shape[0], 128), jnp.float32).at[:, :1].set(w3)
    b3p = jnp.zeros((128,), jnp.float32).at[0].set(b3[0])
    out = pl.pallas_call(
        _readout_body,
        out_shape=jax.ShapeDtypeStruct((n, 128), jnp.float32),
        grid=(n // _ROW_BLK,),
        in_specs=[
            pl.BlockSpec((_ROW_BLK, h.shape[1]), lambda i: (i, 0)),
            pl.BlockSpec((_ROW_BLK, eps.shape[1]), lambda i: (i, 0)),
        ] + [pl.BlockSpec(w.shape, lambda i: (0, 0)) for w in
             (wmu, bmu.reshape(1, -1), wlv, blv.reshape(1, -1),
              w1, b1.reshape(1, -1), w2, b2.reshape(1, -1),
              w3p, b3p.reshape(1, -1))],
        out_specs=pl.BlockSpec((_ROW_BLK, 128), lambda i: (i, 0)),
    )(h, eps, wmu, bmu.reshape(1, -1), wlv, blv.reshape(1, -1),
      w1, b1.reshape(1, -1), w2, b2.reshape(1, -1), w3p, b3p.reshape(1, -1))
    return out[:, 0]


def _seg(seg_fn, table, src_t, dst_t, n_dst_pad):
    flat, bndr, bndi = seg_fn(table, src_t, dst_t)
    m = flat.reshape(n_dst_pad, HID)
    return _bnd_combine(m, bndr, bndi[:, 0].reshape(1, N_TILES))


def kernel(path_features, link_features, eps, path_indices, link_indices,
           W_path_in, b_path_in, W_link_in, b_link_in,
           Wi_p, Wh_p, bi_p, bh_p, Wi_l, Wh_l, bi_l, bh_l,
           W_mu, b_mu, W_lv, b_lv, W_r1, b_r1, W_r2, b_r2, W_r3, b_r3):
    pidx = path_indices.astype(jnp.int32)
    lidx = link_indices.astype(jnp.int32)

    lidx_s, pidx_s = lax.sort((lidx, pidx), num_keys=1, is_stable=True)

    mp_src, mp_dst = _tile_edges(lidx, pidx, N_PATHS)
    ml_src, ml_dst = _tile_edges(pidx_s, lidx_s, N_LINKS)

    dp = path_features.shape[1]
    dl = link_features.shape[1]
    pf = jnp.zeros((NP_PAD, 128), jnp.float32).at[:N_PATHS, :dp].set(path_features)
    lf = jnp.zeros((NL_PAD, 128), jnp.float32).at[:N_LINKS, :dl].set(link_features)
    wp = jnp.zeros((128, 128), jnp.float32).at[:dp].set(W_path_in)
    wl = jnp.zeros((128, 128), jnp.float32).at[:dl].set(W_link_in)
    eps_p = jnp.zeros((NP_PAD, eps.shape[1]), jnp.float32).at[:N_PATHS].set(eps)

    h_path = _embed(pf, wp, b_path_in)
    h_link = _embed(lf, wl, b_link_in)

    for _ in range(T):
        m_path = _seg(_seg_sum_paths, h_link, mp_src, mp_dst, NP_PAD)
        h_path = _gru(m_path, h_path, Wi_p, Wh_p, bi_p, bh_p)
        m_link = _seg(_seg_sum_links, h_path, ml_src, ml_dst, NL_PAD)
        h_link = _gru(m_link, h_link, Wi_l, Wh_l, bi_l, bh_l)

    pred = _readout(h_path, eps_p, W_mu, b_mu, W_lv, b_lv,
                    W_r1, b_r1, W_r2, b_r2, W_r3, b_r3)
    return pred[:N_PATHS]


# speedup vs baseline: 2.3616x; 1.1678x over previous
import numpy as np
import jax
import jax.numpy as jnp
from jax import lax
from jax.experimental import pallas as pl
from jax.experimental.pallas import tpu as pltpu
from jax.experimental.pallas import tpu_sc as plsc

N_PATHS = 10000
N_LINKS = 5000
N_PL = 320000
HID = 128
T = 8

NP_PAD = 10240
NL_PAD = 5120
CL = 128
CHUNKS = 80
N_TILES = 32
EPT = CHUNKS * CL
W = 560

_HALF = [10080] * 11 + [9840] * 4 + [9760]
WINDOW_SIZES = np.array(_HALF + _HALF, np.int32)
WINDOW_STARTS = np.concatenate([[0], np.cumsum(WINDOW_SIZES)[:-1]]).astype(np.int32)
assert int(WINDOW_SIZES.sum()) == N_PL

_ROW_BLK = 1024


def _make_seg_sum(n_dst_pad, sent):
    mesh = plsc.VectorSubcoreMesh(core_axis_name="c", subcore_axis_name="s")

    def body(table_hbm, sidx_hbm, didx_hbm, out_hbm, bndr_hbm, bndi_hbm,
             sidx_v, didx_v, rows_v, rows_b, dense_v, nb_v, sem, sem_b):
        c = lax.axis_index("c")
        s = lax.axis_index("s")
        wid = c * 16 + s
        pltpu.sync_copy(sidx_hbm.at[wid], sidx_v)
        pltpu.sync_copy(didx_hbm.at[wid], didx_v.at[pl.ds(0, EPT)])

        zero16 = jnp.zeros((16,), jnp.float32)
        lane = lax.iota(jnp.int32, 16)

        @pl.loop(0, W * (HID // 16))
        def _zero(r):
            dense_v[pl.ds(r * 16, 16)] = zero16

        base0 = didx_v[pl.ds(0, 16)][0]
        base = jnp.where(wid == 0, 0, base0)

        @pl.when(wid < N_TILES - 1)
        def _load_next():
            pltpu.sync_copy(didx_hbm.at[wid + 1].at[pl.ds(0, 16)], nb_v)

        nxt = nb_v[pl.ds(0, 16)][0]
        range_end = jnp.where(wid == N_TILES - 1, sent, nxt)
        span = range_end - base

        def fold_chunk(q, buf, carry):
            def edge(e, carry):
                cur, acc = carry
                d = didx_v[pl.ds(q * CL + e, 16)][0]
                is_new = d != cur

                @pl.when(jnp.logical_and(is_new, cur >= 0))
                def _store_run():
                    slot = jnp.minimum(jnp.maximum(cur - base, 0), W - 1) * HID
                    for v in range(HID // 16):
                        dense_v[pl.ds(slot + v * 16, 16)] = acc[v]

                acc2 = tuple(
                    jnp.where(is_new, buf[e, pl.ds(v * 16, 16)],
                              acc[v] + buf[e, pl.ds(v * 16, 16)])
                    for v in range(HID // 16))
                return (d, acc2)

            return pl.loop(0, CL, init_carry=carry)(edge)

        carry0 = (jnp.int32(-1), tuple(zero16 for _ in range(8)))
        pltpu.async_copy(table_hbm.at[sidx_v.at[0]], rows_v, sem)

        def pair(p, carry):
            pltpu.make_async_copy(table_hbm.at[sidx_v.at[2 * p]], rows_v, sem).wait()
            pltpu.async_copy(table_hbm.at[sidx_v.at[2 * p + 1]], rows_b, sem_b)
            carry = fold_chunk(2 * p, rows_v, carry)
            pltpu.make_async_copy(table_hbm.at[sidx_v.at[2 * p + 1]], rows_b, sem_b).wait()

            @pl.when(p < CHUNKS // 2 - 1)
            def _fire_next():
                pltpu.async_copy(table_hbm.at[sidx_v.at[2 * p + 2]], rows_v, sem)

            return fold_chunk(2 * p + 1, rows_b, carry)

        pl.loop(0, CHUNKS // 2, init_carry=carry0)(pair)

        nblk = span // 16

        @pl.loop(0, nblk)
        def _wb(k):
            pltpu.sync_copy(dense_v.at[pl.ds(k * 16 * HID, 16 * HID)],
                            out_hbm.at[pl.ds((base + k * 16) * HID, 16 * HID)])

        @pl.loop(nblk * 16, span)
        def _wb_tail(r):
            pltpu.sync_copy(dense_v.at[pl.ds(r * HID, HID)],
                            out_hbm.at[pl.ds((base + r) * HID, HID)])

        bslot = jnp.minimum(span, W - 1) * HID
        pltpu.sync_copy(dense_v.at[pl.ds(bslot, HID)], bndr_hbm.at[wid])
        nb_v[pl.ds(0, 16)] = jnp.where(lane == 0,
                                       jnp.full((16,), range_end, jnp.int32),
                                       jnp.zeros((16,), jnp.int32))
        pltpu.sync_copy(nb_v, bndi_hbm.at[wid].at[pl.ds(0, 16)])

    return pl.kernel(
        body,
        out_type=(jax.ShapeDtypeStruct((n_dst_pad * HID,), jnp.float32),
                  jax.ShapeDtypeStruct((N_TILES, HID), jnp.float32),
                  jax.ShapeDtypeStruct((N_TILES, 16), jnp.int32)),
        mesh=mesh,
        scratch_types=[
            pltpu.VMEM((CHUNKS, CL), jnp.int32),
            pltpu.VMEM((EPT + 16,), jnp.int32),
            pltpu.VMEM((CL, HID), jnp.float32),
            pltpu.VMEM((CL, HID), jnp.float32),
            pltpu.VMEM((W * HID,), jnp.float32),
            pltpu.VMEM((16,), jnp.int32),
            pltpu.SemaphoreType.DMA,
            pltpu.SemaphoreType.DMA,
        ],
    )


_seg_sum_paths = _make_seg_sum(NP_PAD, N_PATHS)
_seg_sum_links = _make_seg_sum(NL_PAD, N_LINKS)


def _tile_edges(src, dst, sent):
    col = np.arange(EPT, dtype=np.int32)[None, :]
    valid = jnp.asarray(col < WINDOW_SIZES[:, None])
    gidx = jnp.asarray(np.minimum(WINDOW_STARTS[:, None] + col, N_PL - 1))
    src_t = jnp.where(valid, src[gidx], 0).reshape(N_TILES, CHUNKS, CL)
    dst_t = jnp.where(valid, dst[gidx], sent).reshape(N_TILES, EPT)
    return src_t, dst_t


def _bnd_body(m_ref, bndr_ref, bndi_ref, o_ref):
    i = pl.program_id(0)

    @pl.when(i == 0)
    def _init():
        o_ref[...] = m_ref[...]

    rid = bndi_ref[0, i]

    @pl.when(rid < o_ref.shape[0])
    def _add():
        o_ref[pl.ds(rid, 1), :] = o_ref[pl.ds(rid, 1), :] + bndr_ref[pl.ds(i, 1), :]


def _bnd_combine(m, bndr, bndi):
    n = m.shape[0]
    return pl.pallas_call(
        _bnd_body,
        out_shape=jax.ShapeDtypeStruct((n, HID), jnp.float32),
        grid=(N_TILES,),
        in_specs=[
            pl.BlockSpec((n, HID), lambda i: (0, 0)),
            pl.BlockSpec((N_TILES, HID), lambda i: (0, 0)),
            pl.BlockSpec(memory_space=pltpu.SMEM),
        ],
        out_specs=pl.BlockSpec((n, HID), lambda i: (0, 0)),
    )(m, bndr, bndi)


def _embed_body(x_ref, w_ref, b_ref, o_ref):
    o_ref[...] = jnp.tanh(
        jnp.dot(x_ref[...], w_ref[...], preferred_element_type=jnp.float32)
        + b_ref[...])


def _embed(x, w, b):
    n, d = x.shape
    p = w.shape[1]
    return pl.pallas_call(
        _embed_body,
        out_shape=jax.ShapeDtypeStruct((n, p), jnp.float32),
        grid=(n // _ROW_BLK,),
        in_specs=[
            pl.BlockSpec((_ROW_BLK, d), lambda i: (i, 0)),
            pl.BlockSpec((d, p), lambda i: (0, 0)),
            pl.BlockSpec((1, p), lambda i: (0, 0)),
        ],
        out_specs=pl.BlockSpec((_ROW_BLK, p), lambda i: (i, 0)),
    )(x, w, b.reshape(1, -1))


def _gru_body(m_ref, h_ref, wi_ref, wh_ref, bi_ref, bh_ref, o_ref):
    m = m_ref[...]
    h = h_ref[...]
    gi = jnp.dot(m, wi_ref[...], preferred_element_type=jnp.float32) + bi_ref[...]
    gh = jnp.dot(h, wh_ref[...], preferred_element_type=jnp.float32) + bh_ref[...]
    H = h.shape[1]
    r = jax.nn.sigmoid(gi[:, :H] + gh[:, :H])
    z = jax.nn.sigmoid(gi[:, H:2 * H] + gh[:, H:2 * H])
    n = jnp.tanh(gi[:, 2 * H:] + r * gh[:, 2 * H:])
    o_ref[...] = (1.0 - z) * n + z * h


def _gru(m, h, wi, wh, bi, bh):
    n, H = h.shape
    return pl.pallas_call(
        _gru_body,
        out_shape=jax.ShapeDtypeStruct((n, H), jnp.float32),
        grid=(n // _ROW_BLK,),
        in_specs=[
            pl.BlockSpec((_ROW_BLK, H), lambda i: (i, 0)),
            pl.BlockSpec((_ROW_BLK, H), lambda i: (i, 0)),
            pl.BlockSpec((H, 3 * H), lambda i: (0, 0)),
            pl.BlockSpec((H, 3 * H), lambda i: (0, 0)),
            pl.BlockSpec((1, 3 * H), lambda i: (0, 0)),
            pl.BlockSpec((1, 3 * H), lambda i: (0, 0)),
        ],
        out_specs=pl.BlockSpec((_ROW_BLK, H), lambda i: (i, 0)),
    )(m, h, wi, wh, bi.reshape(1, -1), bh.reshape(1, -1))


def _readout_body(h_ref, eps_ref, wmu_ref, bmu_ref, wlv_ref, blv_ref,
                  w1_ref, b1_ref, w2_ref, b2_ref, w3_ref, b3_ref, o_ref):
    h = h_ref[...]
    mu = jnp.dot(h, wmu_ref[...], preferred_element_type=jnp.float32) + bmu_ref[...]
    lv = jnp.dot(h, wlv_ref[...], preferred_element_type=jnp.float32) + blv_ref[...]
    z = eps_ref[...] * jnp.exp(0.5 * lv) + mu
    a = jnp.maximum(
        jnp.dot(z, w1_ref[...], preferred_element_type=jnp.float32) + b1_ref[...], 0.0)
    a = jnp.maximum(
        jnp.dot(a, w2_ref[...], preferred_element_type=jnp.float32) + b2_ref[...], 0.0)
    o_ref[...] = jnp.dot(a, w3_ref[...], preferred_element_type=jnp.float32) + b3_ref[...]


def _readout(h, eps, wmu, bmu, wlv, blv, w1, b1, w2, b2, w3, b3):
    n = h.shape[0]
    w3p = jnp.zeros((w3.shape[0], 128), jnp.float32).at[:, :1].set(w3)
    b3p = jnp.zeros((128,), jnp.float32).at[0].set(b3[0])
    out = pl.pallas_call(
        _readout_body,
        out_shape=jax.ShapeDtypeStruct((n, 128), jnp.float32),
        grid=(n // _ROW_BLK,),
        in_specs=[
            pl.BlockSpec((_ROW_BLK, h.shape[1]), lambda i: (i, 0)),
            pl.BlockSpec((_ROW_BLK, eps.shape[1]), lambda i: (i, 0)),
        ] + [pl.BlockSpec(w.shape, lambda i: (0, 0)) for w in
             (wmu, bmu.reshape(1, -1), wlv, blv.reshape(1, -1),
              w1, b1.reshape(1, -1), w2, b2.reshape(1, -1),
              w3p, b3p.reshape(1, -1))],
        out_specs=pl.BlockSpec((_ROW_BLK, 128), lambda i: (i, 0)),
    )(h, eps, wmu, bmu.reshape(1, -1), wlv, blv.reshape(1, -1),
      w1, b1.reshape(1, -1), w2, b2.reshape(1, -1), w3p, b3p.reshape(1, -1))
    return out[:, 0]


def _seg(seg_fn, table, src_t, dst_t, n_dst_pad):
    flat, bndr, bndi = seg_fn(table, src_t, dst_t)
    m = flat.reshape(n_dst_pad, HID)
    return _bnd_combine(m, bndr, bndi[:, 0].reshape(1, N_TILES))


def kernel(path_features, link_features, eps, path_indices, link_indices,
           W_path_in, b_path_in, W_link_in, b_link_in,
           Wi_p, Wh_p, bi_p, bh_p, Wi_l, Wh_l, bi_l, bh_l,
           W_mu, b_mu, W_lv, b_lv, W_r1, b_r1, W_r2, b_r2, W_r3, b_r3):
    pidx = path_indices.astype(jnp.int32)
    lidx = link_indices.astype(jnp.int32)

    lidx_s, pidx_s = lax.sort((lidx, pidx), num_keys=1, is_stable=True)

    mp_src, mp_dst = _tile_edges(lidx, pidx, N_PATHS)
    ml_src, ml_dst = _tile_edges(pidx_s, lidx_s, N_LINKS)

    dp = path_features.shape[1]
    dl = link_features.shape[1]
    pf = jnp.zeros((NP_PAD, 128), jnp.float32).at[:N_PATHS, :dp].set(path_features)
    lf = jnp.zeros((NL_PAD, 128), jnp.float32).at[:N_LINKS, :dl].set(link_features)
    wp = jnp.zeros((128, 128), jnp.float32).at[:dp].set(W_path_in)
    wl = jnp.zeros((128, 128), jnp.float32).at[:dl].set(W_link_in)
    eps_p = jnp.zeros((NP_PAD, eps.shape[1]), jnp.float32).at[:N_PATHS].set(eps)

    h_path = _embed(pf, wp, b_path_in)
    h_link = _embed(lf, wl, b_link_in)

    for _ in range(T):
        m_path = _seg(_seg_sum_paths, h_link, mp_src, mp_dst, NP_PAD)
        h_path = _gru(m_path, h_path, Wi_p, Wh_p, bi_p, bh_p)
        m_link = _seg(_seg_sum_links, h_path, ml_src, ml_dst, NL_PAD)
        h_link = _gru(m_link, h_link, Wi_l, Wh_l, bi_l, bh_l)

    pred = _readout(h_path, eps_p, W_mu, b_mu, W_lv, b_lv,
                    W_r1, b_r1, W_r2, b_r2, W_r3, b_r3)
    return pred[:N_PATHS]


# speedup vs baseline: 2.4123x; 1.0215x over previous
import numpy as np
import jax
import jax.numpy as jnp
from jax import lax
from jax.experimental import pallas as pl
from jax.experimental.pallas import tpu as pltpu
from jax.experimental.pallas import tpu_sc as plsc

N_PATHS = 10000
N_LINKS = 5000
N_PL = 320000
HID = 128
T = 8

NP_PAD = 10240
NL_PAD = 5120
CL = 128
CHUNKS = 80
N_TILES = 32
EPT = CHUNKS * CL
W = 560

_HALF = [10080] * 11 + [9840] * 4 + [9760]
WINDOW_SIZES = np.array(_HALF + _HALF, np.int32)
WINDOW_STARTS = np.concatenate([[0], np.cumsum(WINDOW_SIZES)[:-1]]).astype(np.int32)
assert int(WINDOW_SIZES.sum()) == N_PL

_ROW_BLK = 1024


def _make_seg_sum(n_dst_pad, sent):
    mesh = plsc.VectorSubcoreMesh(core_axis_name="c", subcore_axis_name="s")

    def body(table_hbm, sidx_hbm, didx_hbm, out_hbm, bndr_hbm, bndi_hbm,
             sidx_v, didx_v, rows_v, rows_b, dense_v, nb_v, sem, sem_b):
        c = lax.axis_index("c")
        s = lax.axis_index("s")
        wid = c * 16 + s
        pltpu.sync_copy(sidx_hbm.at[wid], sidx_v)
        pltpu.sync_copy(didx_hbm.at[wid], didx_v.at[pl.ds(0, EPT)])

        zero16 = jnp.zeros((16,), jnp.float32)
        lane = lax.iota(jnp.int32, 16)

        @pl.loop(0, W * (HID // 16), unroll=8)
        def _zero(r):
            dense_v[pl.ds(r * 16, 16)] = zero16

        base0 = didx_v[pl.ds(0, 16)][0]
        base = jnp.where(wid == 0, 0, base0)

        @pl.when(wid < N_TILES - 1)
        def _load_next():
            pltpu.sync_copy(didx_hbm.at[wid + 1].at[pl.ds(0, 16)], nb_v)

        nxt = nb_v[pl.ds(0, 16)][0]
        range_end = jnp.where(wid == N_TILES - 1, sent, nxt)
        span = range_end - base

        def fold_chunk(q, buf, carry):
            def edge(e, carry):
                cur, acc = carry
                d = didx_v[pl.ds(q * CL + e, 16)][0]
                is_new = d != cur

                @pl.when(jnp.logical_and(is_new, cur >= 0))
                def _store_run():
                    slot = jnp.minimum(jnp.maximum(cur - base, 0), W - 1) * HID
                    for v in range(HID // 16):
                        dense_v[pl.ds(slot + v * 16, 16)] = acc[v]

                acc2 = tuple(
                    jnp.where(is_new, buf[e, pl.ds(v * 16, 16)],
                              acc[v] + buf[e, pl.ds(v * 16, 16)])
                    for v in range(HID // 16))
                return (d, acc2)

            return pl.loop(0, CL, init_carry=carry, unroll=8)(edge)

        carry0 = (jnp.int32(-1), tuple(zero16 for _ in range(8)))
        pltpu.async_copy(table_hbm.at[sidx_v.at[0]], rows_v, sem)

        def pair(p, carry):
            pltpu.make_async_copy(table_hbm.at[sidx_v.at[2 * p]], rows_v, sem).wait()
            pltpu.async_copy(table_hbm.at[sidx_v.at[2 * p + 1]], rows_b, sem_b)
            carry = fold_chunk(2 * p, rows_v, carry)
            pltpu.make_async_copy(table_hbm.at[sidx_v.at[2 * p + 1]], rows_b, sem_b).wait()

            @pl.when(p < CHUNKS // 2 - 1)
            def _fire_next():
                pltpu.async_copy(table_hbm.at[sidx_v.at[2 * p + 2]], rows_v, sem)

            return fold_chunk(2 * p + 1, rows_b, carry)

        pl.loop(0, CHUNKS // 2, init_carry=carry0)(pair)

        nblk = span // 16

        @pl.loop(0, nblk)
        def _wb(k):
            pltpu.sync_copy(dense_v.at[pl.ds(k * 16 * HID, 16 * HID)],
                            out_hbm.at[pl.ds((base + k * 16) * HID, 16 * HID)])

        @pl.loop(nblk * 16, span)
        def _wb_tail(r):
            pltpu.sync_copy(dense_v.at[pl.ds(r * HID, HID)],
                            out_hbm.at[pl.ds((base + r) * HID, HID)])

        bslot = jnp.minimum(span, W - 1) * HID
        pltpu.sync_copy(dense_v.at[pl.ds(bslot, HID)], bndr_hbm.at[wid])
        nb_v[pl.ds(0, 16)] = jnp.where(lane == 0,
                                       jnp.full((16,), range_end, jnp.int32),
                                       jnp.zeros((16,), jnp.int32))
        pltpu.sync_copy(nb_v, bndi_hbm.at[wid].at[pl.ds(0, 16)])

    return pl.kernel(
        body,
        out_type=(jax.ShapeDtypeStruct((n_dst_pad * HID,), jnp.float32),
                  jax.ShapeDtypeStruct((N_TILES, HID), jnp.float32),
                  jax.ShapeDtypeStruct((N_TILES, 16), jnp.int32)),
        mesh=mesh,
        scratch_types=[
            pltpu.VMEM((CHUNKS, CL), jnp.int32),
            pltpu.VMEM((EPT + 16,), jnp.int32),
            pltpu.VMEM((CL, HID), jnp.float32),
            pltpu.VMEM((CL, HID), jnp.float32),
            pltpu.VMEM((W * HID,), jnp.float32),
            pltpu.VMEM((16,), jnp.int32),
            pltpu.SemaphoreType.DMA,
            pltpu.SemaphoreType.DMA,
        ],
    )


_seg_sum_paths = _make_seg_sum(NP_PAD, N_PATHS)
_seg_sum_links = _make_seg_sum(NL_PAD, N_LINKS)


def _tile_edges(src, dst, sent):
    col = np.arange(EPT, dtype=np.int32)[None, :]
    valid = jnp.asarray(col < WINDOW_SIZES[:, None])
    gidx = jnp.asarray(np.minimum(WINDOW_STARTS[:, None] + col, N_PL - 1))
    src_t = jnp.where(valid, src[gidx], 0).reshape(N_TILES, CHUNKS, CL)
    dst_t = jnp.where(valid, dst[gidx], sent).reshape(N_TILES, EPT)
    return src_t, dst_t


def _bnd_body(m_ref, bndr_ref, bndi_ref, o_ref):
    i = pl.program_id(0)

    @pl.when(i == 0)
    def _init():
        o_ref[...] = m_ref[...]

    rid = bndi_ref[0, i]

    @pl.when(rid < o_ref.shape[0])
    def _add():
        o_ref[pl.ds(rid, 1), :] = o_ref[pl.ds(rid, 1), :] + bndr_ref[pl.ds(i, 1), :]


def _bnd_combine(m, bndr, bndi):
    n = m.shape[0]
    return pl.pallas_call(
        _bnd_body,
        out_shape=jax.ShapeDtypeStruct((n, HID), jnp.float32),
        grid=(N_TILES,),
        in_specs=[
            pl.BlockSpec((n, HID), lambda i: (0, 0)),
            pl.BlockSpec((N_TILES, HID), lambda i: (0, 0)),
            pl.BlockSpec(memory_space=pltpu.SMEM),
        ],
        out_specs=pl.BlockSpec((n, HID), lambda i: (0, 0)),
    )(m, bndr, bndi)


def _embed_body(x_ref, w_ref, b_ref, o_ref):
    o_ref[...] = jnp.tanh(
        jnp.dot(x_ref[...], w_ref[...], preferred_element_type=jnp.float32)
        + b_ref[...])


def _embed(x, w, b):
    n, d = x.shape
    p = w.shape[1]
    return pl.pallas_call(
        _embed_body,
        out_shape=jax.ShapeDtypeStruct((n, p), jnp.float32),
        grid=(n // _ROW_BLK,),
        in_specs=[
            pl.BlockSpec((_ROW_BLK, d), lambda i: (i, 0)),
            pl.BlockSpec((d, p), lambda i: (0, 0)),
            pl.BlockSpec((1, p), lambda i: (0, 0)),
        ],
        out_specs=pl.BlockSpec((_ROW_BLK, p), lambda i: (i, 0)),
    )(x, w, b.reshape(1, -1))


def _gru_body(m_ref, h_ref, wi_ref, wh_ref, bi_ref, bh_ref, o_ref):
    m = m_ref[...]
    h = h_ref[...]
    gi = jnp.dot(m, wi_ref[...], preferred_element_type=jnp.float32) + bi_ref[...]
    gh = jnp.dot(h, wh_ref[...], preferred_element_type=jnp.float32) + bh_ref[...]
    H = h.shape[1]
    r = jax.nn.sigmoid(gi[:, :H] + gh[:, :H])
    z = jax.nn.sigmoid(gi[:, H:2 * H] + gh[:, H:2 * H])
    n = jnp.tanh(gi[:, 2 * H:] + r * gh[:, 2 * H:])
    o_ref[...] = (1.0 - z) * n + z * h


def _gru(m, h, wi, wh, bi, bh):
    n, H = h.shape
    return pl.pallas_call(
        _gru_body,
        out_shape=jax.ShapeDtypeStruct((n, H), jnp.float32),
        grid=(n // _ROW_BLK,),
        in_specs=[
            pl.BlockSpec((_ROW_BLK, H), lambda i: (i, 0)),
            pl.BlockSpec((_ROW_BLK, H), lambda i: (i, 0)),
            pl.BlockSpec((H, 3 * H), lambda i: (0, 0)),
            pl.BlockSpec((H, 3 * H), lambda i: (0, 0)),
            pl.BlockSpec((1, 3 * H), lambda i: (0, 0)),
            pl.BlockSpec((1, 3 * H), lambda i: (0, 0)),
        ],
        out_specs=pl.BlockSpec((_ROW_BLK, H), lambda i: (i, 0)),
    )(m, h, wi, wh, bi.reshape(1, -1), bh.reshape(1, -1))


def _readout_body(h_ref, eps_ref, wmu_ref, bmu_ref, wlv_ref, blv_ref,
                  w1_ref, b1_ref, w2_ref, b2_ref, w3_ref, b3_ref, o_ref):
    h = h_ref[...]
    mu = jnp.dot(h, wmu_ref[...], preferred_element_type=jnp.float32) + bmu_ref[...]
    lv = jnp.dot(h, wlv_ref[...], preferred_element_type=jnp.float32) + blv_ref[...]
    z = eps_ref[...] * jnp.exp(0.5 * lv) + mu
    a = jnp.maximum(
        jnp.dot(z, w1_ref[...], preferred_element_type=jnp.float32) + b1_ref[...], 0.0)
    a = jnp.maximum(
        jnp.dot(a, w2_ref[...], preferred_element_type=jnp.float32) + b2_ref[...], 0.0)
    o_ref[...] = jnp.dot(a, w3_ref[...], preferred_element_type=jnp.float32) + b3_ref[...]


def _readout(h, eps, wmu, bmu, wlv, blv, w1, b1, w2, b2, w3, b3):
    n = h.shape[0]
    w3p = jnp.zeros((w3.shape[0], 128), jnp.float32).at[:, :1].set(w3)
    b3p = jnp.zeros((128,), jnp.float32).at[0].set(b3[0])
    out = pl.pallas_call(
        _readout_body,
        out_shape=jax.ShapeDtypeStruct((n, 128), jnp.float32),
        grid=(n // _ROW_BLK,),
        in_specs=[
            pl.BlockSpec((_ROW_BLK, h.shape[1]), lambda i: (i, 0)),
            pl.BlockSpec((_ROW_BLK, eps.shape[1]), lambda i: (i, 0)),
        ] + [pl.BlockSpec(w.shape, lambda i: (0, 0)) for w in
             (wmu, bmu.reshape(1, -1), wlv, blv.reshape(1, -1),
              w1, b1.reshape(1, -1), w2, b2.reshape(1, -1),
              w3p, b3p.reshape(1, -1))],
        out_specs=pl.BlockSpec((_ROW_BLK, 128), lambda i: (i, 0)),
    )(h, eps, wmu, bmu.reshape(1, -1), wlv, blv.reshape(1, -1),
      w1, b1.reshape(1, -1), w2, b2.reshape(1, -1), w3p, b3p.reshape(1, -1))
    return out[:, 0]


def _seg(seg_fn, table, src_t, dst_t, n_dst_pad):
    flat, bndr, bndi = seg_fn(table, src_t, dst_t)
    m = flat.reshape(n_dst_pad, HID)
    return _bnd_combine(m, bndr, bndi[:, 0].reshape(1, N_TILES))


def kernel(path_features, link_features, eps, path_indices, link_indices,
           W_path_in, b_path_in, W_link_in, b_link_in,
           Wi_p, Wh_p, bi_p, bh_p, Wi_l, Wh_l, bi_l, bh_l,
           W_mu, b_mu, W_lv, b_lv, W_r1, b_r1, W_r2, b_r2, W_r3, b_r3):
    pidx = path_indices.astype(jnp.int32)
    lidx = link_indices.astype(jnp.int32)

    lidx_s, pidx_s = lax.sort((lidx, pidx), num_keys=1, is_stable=True)

    mp_src, mp_dst = _tile_edges(lidx, pidx, N_PATHS)
    ml_src, ml_dst = _tile_edges(pidx_s, lidx_s, N_LINKS)

    dp = path_features.shape[1]
    dl = link_features.shape[1]
    pf = jnp.zeros((NP_PAD, 128), jnp.float32).at[:N_PATHS, :dp].set(path_features)
    lf = jnp.zeros((NL_PAD, 128), jnp.float32).at[:N_LINKS, :dl].set(link_features)
    wp = jnp.zeros((128, 128), jnp.float32).at[:dp].set(W_path_in)
    wl = jnp.zeros((128, 128), jnp.float32).at[:dl].set(W_link_in)
    eps_p = jnp.zeros((NP_PAD, eps.shape[1]), jnp.float32).at[:N_PATHS].set(eps)

    h_path = _embed(pf, wp, b_path_in)
    h_link = _embed(lf, wl, b_link_in)

    for _ in range(T):
        m_path = _seg(_seg_sum_paths, h_link, mp_src, mp_dst, NP_PAD)
        h_path = _gru(m_path, h_path, Wi_p, Wh_p, bi_p, bh_p)
        m_link = _seg(_seg_sum_links, h_path, ml_src, ml_dst, NL_PAD)
        h_link = _gru(m_link, h_link, Wi_l, Wh_l, bi_l, bh_l)

    pred = _readout(h_path, eps_p, W_mu, b_mu, W_lv, b_lv,
                    W_r1, b_r1, W_r2, b_r2, W_r3, b_r3)
    return pred[:N_PATHS]
